# Initial kernel scaffold; baseline (speedup 1.0000x reference)
import jax
import jax.numpy as jnp
from jax.experimental import pallas as pl


def kernel(x, rows0, cols0, vals0, rows1, cols1, vals1, bias):
    raise NotImplementedError("write your pallas kernel here")


# speedup vs baseline: 7.8391x; 7.8391x over previous
import dataclasses
import functools

import jax
import jax.numpy as jnp
from jax import lax
from jax.experimental import pallas as pl
from jax.experimental.pallas import tpu as pltpu
from jax.experimental.pallas import tpu_sc as plsc

N = 16384
B = 64
NNZ = 268435

NC = 2
NS = 16
LANES = 16
BH = B // NC

CHUNK = 128
PER_TILE = -(-NNZ // NS)
NCHUNK = -(-PER_TILE // CHUNK)
PER_TILE_PAD = NCHUNK * CHUNK
NNZ_PAD = NS * PER_TILE_PAD

RPT = N // NS


def _splat(i):
    return jnp.zeros((LANES,), jnp.int32) + i


def _compiler_params():
    cp = pltpu.CompilerParams()
    fields = pltpu.CompilerParams.__dataclass_fields__
    if "needs_layout_passes" in fields:
        cp = dataclasses.replace(cp, needs_layout_passes=False)
    if "use_tc_tiling_on_sc" in fields:
        cp = dataclasses.replace(cp, use_tc_tiling_on_sc=False)
    return cp


@functools.cache
def _build_sc_chain():
    @functools.partial(
        pl.kernel,
        out_type=jax.ShapeDtypeStruct((NC, N, BH), jnp.float32),
        mesh=plsc.VectorSubcoreMesh(core_axis_name="core",
                                    subcore_axis_name="subcore"),
        compiler_params=_compiler_params(),
        scratch_types=[
            pltpu.VMEM((NCHUNK, CHUNK), jnp.int32),
            pltpu.VMEM((NCHUNK, CHUNK), jnp.int32),
            pltpu.VMEM((NCHUNK, CHUNK), jnp.float32),
            pltpu.VMEM((CHUNK, BH), jnp.float32),
            pltpu.VMEM((CHUNK, BH), jnp.float32),
            pltpu.VMEM((CHUNK,), jnp.float32),
            pltpu.VMEM_SHARED((N, BH), jnp.float32),
            pltpu.VMEM_SHARED((N, BH), jnp.float32),
        ],
    )
    def _sc_chain(xcat_hbm, r1_hbm, c1_hbm, v1_hbm, r0_hbm, c0_hbm, v0_hbm,
                  bias_hbm, out_hbm,
                  cols_v, rows_v, vals_v, g_v, obuf, bias_v, acc1, acc0):
        c = lax.axis_index("core")
        s = lax.axis_index("subcore")

        zero = jnp.zeros((LANES,), jnp.float32)

        @pl.loop(0, CHUNK)
        def _(i):
            obuf[i, pl.ds(0, LANES)] = zero
            obuf[i, pl.ds(LANES, LANES)] = zero

        @pl.loop(0, RPT // CHUNK)
        def _(b):
            pltpu.sync_copy(obuf, acc1.at[pl.ds(s * RPT + b * CHUNK, CHUNK)])
            pltpu.sync_copy(obuf, acc0.at[pl.ds(s * RPT + b * CHUNK, CHUNK)])

        plsc.subcore_barrier()

        def load_indices(r_hbm, c_hbm, v_hbm, col_off):
            pltpu.sync_copy(r_hbm.at[s], rows_v)
            pltpu.sync_copy(c_hbm.at[s], cols_v)
            pltpu.sync_copy(v_hbm.at[s], vals_v)

            if col_off is not None:
                @pl.loop(0, NCHUNK)
                def _(j):
                    @pl.loop(0, CHUNK // LANES)
                    def _(t):
                        sl = (j, pl.ds(t * LANES, LANES))
                        cols_v[sl] = cols_v[sl] + col_off

        def spmm_phase(src_ref, dst_ref):
            @pl.loop(0, NCHUNK)
            def _(j):
                pltpu.sync_copy(src_ref.at[cols_v.at[j]], g_v)

                @pl.loop(0, CHUNK)
                def _(k):
                    val = plsc.load_gather(vals_v, [_splat(j), _splat(k)])
                    g_v[k, pl.ds(0, LANES)] = g_v[k, pl.ds(0, LANES)] * val
                    g_v[k, pl.ds(LANES, LANES)] = (
                        g_v[k, pl.ds(LANES, LANES)] * val)

                pltpu.sync_copy(g_v, dst_ref.at[rows_v.at[j]], add=True)

        load_indices(r1_hbm, c1_hbm, v1_hbm, col_off=c * N)
        spmm_phase(xcat_hbm, acc1)
        plsc.subcore_barrier()

        load_indices(r0_hbm, c0_hbm, v0_hbm, col_off=None)
        spmm_phase(acc1, acc0)
        plsc.subcore_barrier()

        @pl.loop(0, RPT // CHUNK)
        def _(b):
            base = s * RPT + b * CHUNK
            pltpu.sync_copy(bias_hbm.at[pl.ds(base, CHUNK)], bias_v)
            pltpu.sync_copy(acc0.at[pl.ds(base, CHUNK)], obuf)

            @pl.loop(0, CHUNK)
            def _(i):
                bv = plsc.load_gather(bias_v, [_splat(i)])
                obuf[i, pl.ds(0, LANES)] = obuf[i, pl.ds(0, LANES)] + bv
                obuf[i, pl.ds(LANES, LANES)] = (
                    obuf[i, pl.ds(LANES, LANES)] + bv)

            pltpu.sync_copy(obuf, out_hbm.at[c, pl.ds(base, CHUNK)])

    return _sc_chain


def _prep_coo(rows, cols, vals):
    pad = NNZ_PAD - NNZ
    rows = jnp.concatenate([rows, jnp.zeros((pad,), jnp.int32)])
    cols = jnp.concatenate([cols, jnp.zeros((pad,), jnp.int32)])
    vals = jnp.concatenate([vals, jnp.zeros((pad,), jnp.float32)])
    shape = (NS, NCHUNK, CHUNK)
    return rows.reshape(shape), cols.reshape(shape), vals.reshape(shape)


def kernel(x, rows0, cols0, vals0, rows1, cols1, vals1, bias):
    xcat = x.reshape(NC, BH, N).transpose(0, 2, 1).reshape(NC * N, BH)
    r1, c1, v1 = _prep_coo(rows1, cols1, vals1)
    r0, c0, v0 = _prep_coo(rows0, cols0, vals0)
    y = _build_sc_chain()(xcat, r1, c1, v1, r0, c0, v0, bias)
    return y.transpose(0, 2, 1).reshape(B, N)


# speedup vs baseline: 8.9453x; 1.1411x over previous
import dataclasses
import functools

import jax
import jax.numpy as jnp
from jax import lax
from jax.experimental import pallas as pl
from jax.experimental.pallas import tpu as pltpu
from jax.experimental.pallas import tpu_sc as plsc

N = 16384
B = 64
NNZ = 268435

NC = 2
NS = 16
LANES = 16
BH = B // NC

CHUNK = 128
PER_TILE = -(-NNZ // NS)
NCHUNK = -(-PER_TILE // CHUNK)
PER_TILE_PAD = NCHUNK * CHUNK
NNZ_PAD = NS * PER_TILE_PAD

RPT = N // NS

NBUF = 3
HALVES = 2
HC = NCHUNK // HALVES
GROUPS = HC // NBUF


def _splat(i):
    return jnp.zeros((LANES,), jnp.int32) + i


def _compiler_params():
    cp = pltpu.CompilerParams()
    fields = pltpu.CompilerParams.__dataclass_fields__
    if "needs_layout_passes" in fields:
        cp = dataclasses.replace(cp, needs_layout_passes=False)
    if "use_tc_tiling_on_sc" in fields:
        cp = dataclasses.replace(cp, use_tc_tiling_on_sc=False)
    return cp


@functools.cache
def _build_sc_chain():
    @functools.partial(
        pl.kernel,
        out_type=jax.ShapeDtypeStruct((NC, N, BH), jnp.float32),
        mesh=plsc.VectorSubcoreMesh(core_axis_name="core",
                                    subcore_axis_name="subcore"),
        compiler_params=_compiler_params(),
        scratch_types=[
            pltpu.VMEM((HC, CHUNK), jnp.int32),
            pltpu.VMEM((HC, CHUNK), jnp.int32),
            pltpu.VMEM((HC, CHUNK), jnp.float32),
            pltpu.VMEM((CHUNK, BH), jnp.float32),
            pltpu.VMEM((CHUNK, BH), jnp.float32),
            pltpu.VMEM((CHUNK, BH), jnp.float32),
            pltpu.VMEM((CHUNK, BH), jnp.float32),
            pltpu.VMEM((CHUNK, BH), jnp.float32),
            pltpu.VMEM((CHUNK, BH), jnp.float32),
            pltpu.VMEM((CHUNK,), jnp.float32),
            pltpu.SemaphoreType.DMA,
            pltpu.SemaphoreType.DMA,
            pltpu.SemaphoreType.DMA,
            pltpu.SemaphoreType.DMA,
            pltpu.SemaphoreType.DMA,
            pltpu.SemaphoreType.DMA,
            pltpu.VMEM_SHARED((N, BH), jnp.float32),
            pltpu.VMEM_SHARED((N, BH), jnp.float32),
        ],
    )
    def _sc_chain(xcat_hbm, r1_hbm, c1_hbm, v1_hbm, r0_hbm, c0_hbm, v0_hbm,
                  bias_hbm, out_hbm,
                  cols_v, rows_v, vals_v,
                  gbuf0, gbuf1, gbuf2, sbuf0, sbuf1, sbuf2, bias_v,
                  sg0, sg1, sg2, ss0, ss1, ss2, acc1, acc0):
        c = lax.axis_index("core")
        s = lax.axis_index("subcore")
        gbuf = (gbuf0, gbuf1, gbuf2)
        sbuf = (sbuf0, sbuf1, sbuf2)
        sg = (sg0, sg1, sg2)
        ss = (ss0, ss1, ss2)

        zero = jnp.zeros((LANES,), jnp.float32)

        @pl.loop(0, CHUNK)
        def _(i):
            sbuf0[i, pl.ds(0, LANES)] = zero
            sbuf0[i, pl.ds(LANES, LANES)] = zero

        @pl.loop(0, RPT // CHUNK)
        def _(b):
            pltpu.sync_copy(sbuf0, acc1.at[pl.ds(s * RPT + b * CHUNK, CHUNK)])
            pltpu.sync_copy(sbuf0, acc0.at[pl.ds(s * RPT + b * CHUNK, CHUNK)])

        plsc.subcore_barrier()

        def scale_chunk(b, j):
            @pl.loop(0, CHUNK)
            def _(k):
                val = plsc.load_gather(vals_v, [_splat(j), _splat(k)])
                sbuf[b][k, pl.ds(0, LANES)] = gbuf[b][k, pl.ds(0, LANES)] * val
                sbuf[b][k, pl.ds(LANES, LANES)] = (
                    gbuf[b][k, pl.ds(LANES, LANES)] * val)

        def spmm_phase(src_ref, dst_ref, r_hbm, c_hbm, v_hbm, col_off):
            for h in range(HALVES):
                pltpu.sync_copy(r_hbm.at[s, pl.ds(h * HC, HC)], rows_v)
                pltpu.sync_copy(c_hbm.at[s, pl.ds(h * HC, HC)], cols_v)
                pltpu.sync_copy(v_hbm.at[s, pl.ds(h * HC, HC)], vals_v)

                if col_off is not None:
                    @pl.loop(0, HC)
                    def _(j):
                        @pl.loop(0, CHUNK // LANES)
                        def _(t):
                            sl = (j, pl.ds(t * LANES, LANES))
                            cols_v[sl] = cols_v[sl] + col_off

                for b in range(NBUF):
                    pltpu.async_copy(src_ref.at[cols_v.at[b]], gbuf[b], sg[b])

                def group_body(g, first, last):
                    for b in range(NBUF):
                        j = g * NBUF + b
                        pltpu.make_async_copy(
                            src_ref.at[cols_v.at[0]], gbuf[b], sg[b]).wait()
                        if not first:
                            pltpu.make_async_copy(
                                sbuf[b], dst_ref.at[rows_v.at[0]],
                                ss[b]).wait()
                        scale_chunk(b, j)
                        if not last:
                            pltpu.async_copy(
                                src_ref.at[cols_v.at[j + NBUF]],
                                gbuf[b], sg[b])
                        pltpu.async_copy(
                            sbuf[b], dst_ref.at[rows_v.at[j]], ss[b],
                            add=True)

                group_body(0, True, False)

                @pl.loop(1, GROUPS - 1)
                def _(g):
                    group_body(g, False, False)

                group_body(GROUPS - 1, False, True)

                for b in range(NBUF):
                    pltpu.make_async_copy(
                        sbuf[b], dst_ref.at[rows_v.at[0]], ss[b]).wait()

        spmm_phase(xcat_hbm, acc1, r1_hbm, c1_hbm, v1_hbm, col_off=c * N)
        plsc.subcore_barrier()

        spmm_phase(acc1, acc0, r0_hbm, c0_hbm, v0_hbm, col_off=None)
        plsc.subcore_barrier()

        @pl.loop(0, RPT // CHUNK)
        def _(b):
            base = s * RPT + b * CHUNK
            pltpu.sync_copy(bias_hbm.at[pl.ds(base, CHUNK)], bias_v)
            pltpu.sync_copy(acc0.at[pl.ds(base, CHUNK)], sbuf0)

            @pl.loop(0, CHUNK)
            def _(i):
                bv = plsc.load_gather(bias_v, [_splat(i)])
                sbuf0[i, pl.ds(0, LANES)] = sbuf0[i, pl.ds(0, LANES)] + bv
                sbuf0[i, pl.ds(LANES, LANES)] = (
                    sbuf0[i, pl.ds(LANES, LANES)] + bv)

            pltpu.sync_copy(sbuf0, out_hbm.at[c, pl.ds(base, CHUNK)])

    return _sc_chain


def _prep_coo(rows, cols, vals):
    pad = NNZ_PAD - NNZ
    rows = jnp.concatenate([rows, jnp.zeros((pad,), jnp.int32)])
    cols = jnp.concatenate([cols, jnp.zeros((pad,), jnp.int32)])
    vals = jnp.concatenate([vals, jnp.zeros((pad,), jnp.float32)])
    shape = (NS, NCHUNK, CHUNK)
    return rows.reshape(shape), cols.reshape(shape), vals.reshape(shape)


def kernel(x, rows0, cols0, vals0, rows1, cols1, vals1, bias):
    xcat = x.reshape(NC, BH, N).transpose(0, 2, 1).reshape(NC * N, BH)
    r1, c1, v1 = _prep_coo(rows1, cols1, vals1)
    r0, c0, v0 = _prep_coo(rows0, cols0, vals0)
    y = _build_sc_chain()(xcat, r1, c1, v1, r0, c0, v0, bias)
    return y.transpose(0, 2, 1).reshape(B, N)
